# baseline (speedup 1.0000x reference)
import jax
import jax.numpy as jnp
from jax import lax
from jax.experimental import pallas as pl
from jax.experimental.pallas import tpu as pltpu
from jax.experimental.pallas import tpu_sc as plsc

_FIN = 128
_FQK = 64
_N = 10000
_E = 320000
_NPAD = 10240
_NC, _NS, _L = 2, 16, 16
_NW = _NC * _NS
_CH = 128
_NCHUNK = _E // _CH
_BASE_CNT = _NCHUNK // _NW
_EXTRA = _NCHUNK - _BASE_CNT * _NW
_LOOP_CH = 80
_SPAN = _LOOP_CH * _CH

_NODES_PER_TILE = _NPAD // _NS
_GROUPS = _CH // _L


def _proj_body(x_ref, w_ref, q_ref, k_ref):
    qk = jnp.dot(x_ref[...], w_ref[...], preferred_element_type=jnp.float32)
    scale = float(_FQK) ** (-0.5)
    q_ref[...] = (qk[:, :_FQK] * scale).astype(jnp.bfloat16)
    k_ref[...] = qk[:, _FQK:].astype(jnp.bfloat16)


def _project(x, W):
    return pl.pallas_call(
        _proj_body,
        out_shape=(
            jax.ShapeDtypeStruct((_N, _FQK), jnp.bfloat16),
            jax.ShapeDtypeStruct((_N, _FQK), jnp.bfloat16),
        ),
    )(x, W)


def _worker_span(wid):
    base = wid * _BASE_CNT + jnp.minimum(wid, _EXTRA)
    cnt = _BASE_CNT + jnp.where(wid < _EXTRA, 1, 0)
    return base, cnt


def _zero_ref(ref, nwords):
    zeros = jnp.zeros((_L,), jnp.float32)

    def body(i, _):
        ref[pl.ds(i * _L, _L)] = zeros
        return 0

    lax.fori_loop(0, nwords // _L, body, 0)


_SC_PARAMS = pltpu.CompilerParams(
    needs_layout_passes=False, use_tc_tiling_on_sc=False)


def _edge_body(q_hbm, k_hbm, ei_hbm,
               out_hbm, p0_hbm, p1_hbm, flag_hbm,
               sidx_v, didx_v, qr0_v, kr0_v, qr1_v, kr1_v,
               expall_v, acc_v, sum2_v, red_v, tot_v, flag_v,
               shared_sp, sem0, sem1):
    cid = lax.axis_index("c")
    sid = lax.axis_index("s")
    wid = sid * _NC + cid
    base, cnt = _worker_span(wid)
    e0 = base * _CH
    main_words = _BASE_CNT * _CH

    @pl.when(sid == 0)
    def _():
        flag_v[pl.ds(0, _L)] = jnp.zeros((_L,), jnp.float32)
        pltpu.sync_copy(flag_v, flag_hbm.at[cid])

    e0c = jnp.minimum(e0, _E - _SPAN)
    loff = e0 - e0c
    d1 = pltpu.async_copy(ei_hbm.at[0, pl.ds(e0c, _SPAN)], sidx_v, sem0)
    d2 = pltpu.async_copy(ei_hbm.at[1, pl.ds(e0c, _SPAN)], didx_v, sem0)
    d1.wait()
    d2.wait()
    _zero_ref(acc_v, _NPAD)

    bufs = ((qr0_v, kr0_v, sem0), (qr1_v, kr1_v, sem1))

    def _off(c):
        return jnp.minimum(loff + c * _CH, _SPAN - _CH)

    def _gather(c, p):
        qr, kr, sem = bufs[p]
        o = _off(c)
        pltpu.async_copy(q_hbm.at[sidx_v.at[pl.ds(o, _CH)]], qr, sem)
        pltpu.async_copy(k_hbm.at[didx_v.at[pl.ds(o, _CH)]], kr, sem)

    _gather(0, 0)
    lane = jnp.arange(_L, dtype=jnp.int32)

    def pair_body(gi, _):
        for p in range(2):
            c = gi * 2 + p
            qr, kr, sem = bufs[p]

            @pl.when(c + 1 < _LOOP_CH)
            def _():
                _gather(c + 1, 1 - p)

            o = _off(c)
            pltpu.make_async_copy(
                q_hbm.at[sidx_v.at[pl.ds(o, _CH)]], qr, sem).wait()
            pltpu.make_async_copy(
                k_hbm.at[didx_v.at[pl.ds(o, _CH)]], kr, sem).wait()

            in_range = c < cnt
            smask = jnp.full((_L,), in_range)
            gbody = _rowwise_groups(qr, kr, sidx_v, expall_v,
                                    acc_v, smask, lane, c, o)
            for g in range(_GROUPS):
                gbody(g, 0)
        return 0

    lax.fori_loop(0, _LOOP_CH // 2, pair_body, 0)

    pltpu.sync_copy(acc_v, shared_sp.at[sid])
    plsc.subcore_barrier()

    nbase = sid * _NODES_PER_TILE
    pltpu.sync_copy(shared_sp.at[:, pl.ds(nbase, _NODES_PER_TILE)], red_v)

    def add_body(j, _):
        sl = pl.ds(j * _L, _L)
        s = red_v[0, sl]
        for r in range(1, _NS):
            s = s + red_v[r, sl]
        tot_v[sl] = s
        return 0

    lax.fori_loop(0, _NODES_PER_TILE // _L, add_body, 0)

    @pl.when(cid == 0)
    def _():
        pltpu.sync_copy(tot_v, p0_hbm.at[pl.ds(nbase, _NODES_PER_TILE)])

    @pl.when(cid == 1)
    def _():
        pltpu.sync_copy(tot_v, p1_hbm.at[pl.ds(nbase, _NODES_PER_TILE)])

    plsc.subcore_barrier()

    @pl.when(sid == 0)
    def _():
        flag_v[pl.ds(0, _L)] = jnp.ones((_L,), jnp.float32)
        pltpu.sync_copy(flag_v, flag_hbm.at[cid])

    def poll_cond(s):
        return s < 15.5

    def poll_body(s):
        pltpu.sync_copy(flag_hbm.at[1 - cid], flag_v)
        return jnp.sum(flag_v[pl.ds(0, _L)])

    lax.while_loop(poll_cond, poll_body, jnp.float32(0.0))

    da = pltpu.async_copy(p0_hbm, acc_v, sem0)
    db = pltpu.async_copy(p1_hbm, sum2_v, sem1)
    da.wait()
    db.wait()

    def comb_body(j, _):
        sl = pl.ds(j * _L, _L)
        acc_v[sl] = acc_v[sl] + sum2_v[sl]
        return 0

    lax.fori_loop(0, _NPAD // _L, comb_body, 0)

    def div_body(gg, _):
        for u in range(2):
            g = gg * 2 + u
            sl = pl.ds(g * _L, _L)
            srcv = sidx_v[pl.ds(loff + g * _L, _L)]
            sv = plsc.load_gather(acc_v, [srcv])
            sum2_v[sl] = expall_v[sl] / sv
        return 0

    lax.fori_loop(0, cnt * _GROUPS // 2, div_body, 0)

    pltpu.sync_copy(sum2_v.at[pl.ds(0, main_words)],
                    out_hbm.at[pl.ds(e0, main_words)])

    @pl.when(cnt == _BASE_CNT + 1)
    def _():
        pltpu.sync_copy(sum2_v.at[pl.ds(main_words, _CH)],
                        out_hbm.at[pl.ds(e0 + main_words, _CH)])


def _rowwise_groups(qr, kr, sidx_v, expall_v, acc_v, smask, lane, c, o):
    def group_body(g, carry):
        dots = jnp.zeros((_L,), jnp.float32)
        for e in range(_L):
            prod = jnp.zeros((_L,), jnp.float32)
            row = g * _L + e
            for j in range(_FQK // (2 * _L)):
                sl = pl.ds(j * 2 * _L, 2 * _L)
                pp = qr[row, sl] * kr[row, sl]
                pa, pb = plsc.unpack(
                    pp, format=plsc.PackFormat.INTERLEAVED,
                    preferred_element_type=jnp.float32)
                prod = prod + pa + pb
            dots = jnp.where(lane == e, jnp.sum(prod), dots)
        ev = jnp.exp(dots)
        expall_v[pl.ds(c * _CH + g * _L, _L)] = ev
        srcv = sidx_v[pl.ds(o + g * _L, _L)]
        plsc.addupdate_scatter(acc_v, [srcv], ev, mask=smask)
        return carry

    return group_body


def _edge_kernel(q, k, ei):
    mesh = plsc.VectorSubcoreMesh(core_axis_name="c", subcore_axis_name="s")
    kfn = pl.kernel(
        _edge_body,
        out_type=(
            jax.ShapeDtypeStruct((_E,), jnp.float32),
            jax.ShapeDtypeStruct((_NPAD,), jnp.float32),
            jax.ShapeDtypeStruct((_NPAD,), jnp.float32),
            jax.ShapeDtypeStruct((_NC, _L), jnp.float32),
        ),
        mesh=mesh,
        compiler_params=_SC_PARAMS,
        scratch_types=(
            pltpu.VMEM((_SPAN,), jnp.int32),
            pltpu.VMEM((_SPAN,), jnp.int32),
            pltpu.VMEM((_CH, _FQK), jnp.bfloat16),
            pltpu.VMEM((_CH, _FQK), jnp.bfloat16),
            pltpu.VMEM((_CH, _FQK), jnp.bfloat16),
            pltpu.VMEM((_CH, _FQK), jnp.bfloat16),
            pltpu.VMEM((_SPAN,), jnp.float32),
            pltpu.VMEM((_NPAD,), jnp.float32),
            pltpu.VMEM((_NPAD,), jnp.float32),
            pltpu.VMEM((_NS, _NODES_PER_TILE), jnp.float32),
            pltpu.VMEM((_NODES_PER_TILE,), jnp.float32),
            pltpu.VMEM((_L,), jnp.float32),
            pltpu.VMEM_SHARED((_NS, _NPAD), jnp.float32),
            pltpu.SemaphoreType.DMA,
            pltpu.SemaphoreType.DMA,
        ),
    )
    out, _, _, _ = kfn(q, k, ei)
    return out


def kernel(x, batch, ei, W):
    del batch
    q, k = _project(x, W)
    return _edge_kernel(q, k, ei)


# speedup vs baseline: 1.3178x; 1.3178x over previous
import jax
import jax.numpy as jnp
from jax import lax
from jax.experimental import pallas as pl
from jax.experimental.pallas import tpu as pltpu
from jax.experimental.pallas import tpu_sc as plsc

_FIN = 128
_FQK = 64
_N = 10000
_E = 320000
_NPAD = 10240
_NC, _NS, _L = 2, 16, 16
_NW = _NC * _NS
_CH = 128
_NCHUNK = _E // _CH
_BASE_CNT = _NCHUNK // _NW
_EXTRA = _NCHUNK - _BASE_CNT * _NW
_LOOP_CH = 80
_SPAN = _LOOP_CH * _CH

_NODES_PER_TILE = _NPAD // _NS
_GROUPS = _CH // _L


def _proj_body(x_ref, w_ref, q_ref, k_ref):
    qk = jnp.dot(x_ref[...], w_ref[...], preferred_element_type=jnp.float32)
    scale = float(_FQK) ** (-0.5)
    q_ref[...] = (qk[:, :_FQK] * scale).astype(jnp.bfloat16)
    k_ref[...] = qk[:, _FQK:].astype(jnp.bfloat16)


def _project(x, W):
    return pl.pallas_call(
        _proj_body,
        out_shape=(
            jax.ShapeDtypeStruct((_N, _FQK), jnp.bfloat16),
            jax.ShapeDtypeStruct((_N, _FQK), jnp.bfloat16),
        ),
    )(x, W)


def _worker_span(wid):
    base = wid * _BASE_CNT + jnp.minimum(wid, _EXTRA)
    cnt = _BASE_CNT + jnp.where(wid < _EXTRA, 1, 0)
    return base, cnt


def _zero_ref(ref, nwords):
    zeros = jnp.zeros((_L,), jnp.float32)

    def body(i, _):
        ref[pl.ds(i * _L, _L)] = zeros
        return 0

    lax.fori_loop(0, nwords // _L, body, 0)


_SC_PARAMS = pltpu.CompilerParams(
    needs_layout_passes=False, use_tc_tiling_on_sc=False)


def _edge_body(q_hbm, k_hbm, ei_hbm,
               out_hbm, p0_hbm, p1_hbm, flag_hbm,
               sidx_v, didx_v, qr0_v, kr0_v, qr1_v, kr1_v,
               expall_v, acc_v, sum2_v, red_v, tot_v, flag_v,
               shared_sp, sem0, sem1):
    cid = lax.axis_index("c")
    sid = lax.axis_index("s")
    wid = sid * _NC + cid
    base, cnt = _worker_span(wid)
    e0 = base * _CH
    main_words = _BASE_CNT * _CH

    @pl.when(sid == 0)
    def _():
        flag_v[pl.ds(0, _L)] = jnp.zeros((_L,), jnp.float32)
        pltpu.sync_copy(flag_v, flag_hbm.at[cid])

    e0c = jnp.minimum(e0, _E - _SPAN)
    loff = e0 - e0c
    d1 = pltpu.async_copy(ei_hbm.at[0, pl.ds(e0c, _SPAN)], sidx_v, sem0)
    d2 = pltpu.async_copy(ei_hbm.at[1, pl.ds(e0c, _SPAN)], didx_v, sem0)
    d1.wait()
    d2.wait()
    _zero_ref(acc_v, _NPAD)

    bufs = ((qr0_v, kr0_v, sem0), (qr1_v, kr1_v, sem1))

    def _off(c):
        return jnp.minimum(loff + c * _CH, _SPAN - _CH)

    def _gather(c, p):
        qr, kr, sem = bufs[p]
        o = _off(c)
        pltpu.async_copy(q_hbm.at[sidx_v.at[pl.ds(o, _CH)]], qr, sem)
        pltpu.async_copy(k_hbm.at[didx_v.at[pl.ds(o, _CH)]], kr, sem)

    _gather(0, 0)
    lane = jnp.arange(_L, dtype=jnp.int32)

    def pair_body(gi, _):
        for p in range(2):
            c = gi * 2 + p
            qr, kr, sem = bufs[p]

            @pl.when(c + 1 < _LOOP_CH)
            def _():
                _gather(c + 1, 1 - p)

            o = _off(c)
            pltpu.make_async_copy(
                q_hbm.at[sidx_v.at[pl.ds(o, _CH)]], qr, sem).wait()
            pltpu.make_async_copy(
                k_hbm.at[didx_v.at[pl.ds(o, _CH)]], kr, sem).wait()

            in_range = c < cnt
            smask = jnp.full((_L,), in_range)
            gbody = _rowwise_groups(qr, kr, sidx_v, expall_v,
                                    acc_v, smask, lane, c, o)

            def gpair(gg, carry):
                gbody(gg * 2, carry)
                gbody(gg * 2 + 1, carry)
                return carry

            lax.fori_loop(0, _GROUPS // 2, gpair, 0)
        return 0

    lax.fori_loop(0, _LOOP_CH // 2, pair_body, 0)

    pltpu.sync_copy(acc_v, shared_sp.at[sid])
    plsc.subcore_barrier()

    nbase = sid * _NODES_PER_TILE
    pltpu.sync_copy(shared_sp.at[:, pl.ds(nbase, _NODES_PER_TILE)], red_v)

    def add_body(j, _):
        sl = pl.ds(j * _L, _L)
        s = red_v[0, sl]
        for r in range(1, _NS):
            s = s + red_v[r, sl]
        tot_v[sl] = s
        return 0

    lax.fori_loop(0, _NODES_PER_TILE // _L, add_body, 0)

    @pl.when(cid == 0)
    def _():
        pltpu.sync_copy(tot_v, p0_hbm.at[pl.ds(nbase, _NODES_PER_TILE)])

    @pl.when(cid == 1)
    def _():
        pltpu.sync_copy(tot_v, p1_hbm.at[pl.ds(nbase, _NODES_PER_TILE)])

    plsc.subcore_barrier()

    @pl.when(sid == 0)
    def _():
        flag_v[pl.ds(0, _L)] = jnp.ones((_L,), jnp.float32)
        pltpu.sync_copy(flag_v, flag_hbm.at[cid])

    def poll_cond(s):
        return s < 15.5

    def poll_body(s):
        pltpu.sync_copy(flag_hbm.at[1 - cid], flag_v)
        return jnp.sum(flag_v[pl.ds(0, _L)])

    lax.while_loop(poll_cond, poll_body, jnp.float32(0.0))

    da = pltpu.async_copy(p0_hbm, acc_v, sem0)
    db = pltpu.async_copy(p1_hbm, sum2_v, sem1)
    da.wait()
    db.wait()

    def comb_body(j, _):
        sl = pl.ds(j * _L, _L)
        acc_v[sl] = acc_v[sl] + sum2_v[sl]
        return 0

    lax.fori_loop(0, _NPAD // _L, comb_body, 0)

    def div_body(gg, _):
        for u in range(2):
            g = gg * 2 + u
            sl = pl.ds(g * _L, _L)
            srcv = sidx_v[pl.ds(loff + g * _L, _L)]
            sv = plsc.load_gather(acc_v, [srcv])
            sum2_v[sl] = expall_v[sl] / sv
        return 0

    lax.fori_loop(0, cnt * _GROUPS // 2, div_body, 0)

    pltpu.sync_copy(sum2_v.at[pl.ds(0, main_words)],
                    out_hbm.at[pl.ds(e0, main_words)])

    @pl.when(cnt == _BASE_CNT + 1)
    def _():
        pltpu.sync_copy(sum2_v.at[pl.ds(main_words, _CH)],
                        out_hbm.at[pl.ds(e0 + main_words, _CH)])


def _rowwise_groups(qr, kr, sidx_v, expall_v, acc_v, smask, lane, c, o):
    def group_body(g, carry):
        dots = jnp.zeros((_L,), jnp.float32)
        for e in range(_L):
            prod = jnp.zeros((_L,), jnp.float32)
            row = g * _L + e
            for j in range(_FQK // (2 * _L)):
                sl = pl.ds(j * 2 * _L, 2 * _L)
                pp = qr[row, sl] * kr[row, sl]
                pa, pb = plsc.unpack(
                    pp, format=plsc.PackFormat.INTERLEAVED,
                    preferred_element_type=jnp.float32)
                prod = prod + pa + pb
            dots = jnp.where(lane == e, jnp.sum(prod), dots)
        ev = jnp.exp(dots)
        expall_v[pl.ds(c * _CH + g * _L, _L)] = ev
        srcv = sidx_v[pl.ds(o + g * _L, _L)]
        plsc.addupdate_scatter(acc_v, [srcv], ev, mask=smask)
        return carry

    return group_body


def _edge_kernel(q, k, ei):
    mesh = plsc.VectorSubcoreMesh(core_axis_name="c", subcore_axis_name="s")
    kfn = pl.kernel(
        _edge_body,
        out_type=(
            jax.ShapeDtypeStruct((_E,), jnp.float32),
            jax.ShapeDtypeStruct((_NPAD,), jnp.float32),
            jax.ShapeDtypeStruct((_NPAD,), jnp.float32),
            jax.ShapeDtypeStruct((_NC, _L), jnp.float32),
        ),
        mesh=mesh,
        compiler_params=_SC_PARAMS,
        scratch_types=(
            pltpu.VMEM((_SPAN,), jnp.int32),
            pltpu.VMEM((_SPAN,), jnp.int32),
            pltpu.VMEM((_CH, _FQK), jnp.bfloat16),
            pltpu.VMEM((_CH, _FQK), jnp.bfloat16),
            pltpu.VMEM((_CH, _FQK), jnp.bfloat16),
            pltpu.VMEM((_CH, _FQK), jnp.bfloat16),
            pltpu.VMEM((_SPAN,), jnp.float32),
            pltpu.VMEM((_NPAD,), jnp.float32),
            pltpu.VMEM((_NPAD,), jnp.float32),
            pltpu.VMEM((_NS, _NODES_PER_TILE), jnp.float32),
            pltpu.VMEM((_NODES_PER_TILE,), jnp.float32),
            pltpu.VMEM((_L,), jnp.float32),
            pltpu.VMEM_SHARED((_NS, _NPAD), jnp.float32),
            pltpu.SemaphoreType.DMA,
            pltpu.SemaphoreType.DMA,
        ),
    )
    out, _, _, _ = kfn(q, k, ei)
    return out


def kernel(x, batch, ei, W):
    del batch
    q, k = _project(x, W)
    return _edge_kernel(q, k, ei)


# speedup vs baseline: 1.3746x; 1.0431x over previous
import jax
import jax.numpy as jnp
from jax import lax
from jax.experimental import pallas as pl
from jax.experimental.pallas import tpu as pltpu
from jax.experimental.pallas import tpu_sc as plsc

_FIN = 128
_FQK = 64
_N = 10000
_E = 320000
_NPAD = 10240
_NC, _NS, _L = 2, 16, 16
_NW = _NC * _NS
_CH = 128
_NCHUNK = _E // _CH
_BASE_CNT = _NCHUNK // _NW
_EXTRA = _NCHUNK - _BASE_CNT * _NW
_LOOP_CH = 80
_SPAN = _LOOP_CH * _CH

_NODES_PER_TILE = _NPAD // _NS
_GROUPS = _CH // _L


def _proj_body(x_ref, w_ref, q_ref, k_ref):
    qk = jnp.dot(x_ref[...], w_ref[...], preferred_element_type=jnp.float32)
    scale = float(_FQK) ** (-0.5)
    q_ref[...] = (qk[:, :_FQK] * scale).astype(jnp.bfloat16)
    k_ref[...] = qk[:, _FQK:].astype(jnp.bfloat16)


def _project(x, W):
    return pl.pallas_call(
        _proj_body,
        out_shape=(
            jax.ShapeDtypeStruct((_N, _FQK), jnp.bfloat16),
            jax.ShapeDtypeStruct((_N, _FQK), jnp.bfloat16),
        ),
    )(x, W)


def _worker_span(wid):
    base = wid * _BASE_CNT + jnp.minimum(wid, _EXTRA)
    cnt = _BASE_CNT + jnp.where(wid < _EXTRA, 1, 0)
    return base, cnt


def _zero_ref(ref, nwords):
    zeros = jnp.zeros((_L,), jnp.float32)

    def body(i, _):
        for u in range(4):
            ref[pl.ds((i * 4 + u) * _L, _L)] = zeros
        return 0

    lax.fori_loop(0, nwords // (4 * _L), body, 0)


_SC_PARAMS = pltpu.CompilerParams(
    needs_layout_passes=False, use_tc_tiling_on_sc=False)


def _edge_body(q_hbm, k_hbm, ei_hbm,
               out_hbm, p0_hbm, p1_hbm, flag_hbm,
               sidx_v, didx_v, qr0_v, kr0_v, qr1_v, kr1_v,
               expall_v, acc_v, sum2_v, red_v, tot_v, flag_v,
               shared_sp, sem0, sem1):
    cid = lax.axis_index("c")
    sid = lax.axis_index("s")
    wid = sid * _NC + cid
    base, cnt = _worker_span(wid)
    e0 = base * _CH
    main_words = _BASE_CNT * _CH

    @pl.when(sid == 0)
    def _():
        flag_v[pl.ds(0, _L)] = jnp.zeros((_L,), jnp.float32)
        pltpu.sync_copy(flag_v, flag_hbm.at[cid])

    e0c = jnp.minimum(e0, _E - _SPAN)
    loff = e0 - e0c
    d1 = pltpu.async_copy(ei_hbm.at[0, pl.ds(e0c, _SPAN)], sidx_v, sem0)
    d2 = pltpu.async_copy(ei_hbm.at[1, pl.ds(e0c, _SPAN)], didx_v, sem0)
    d1.wait()
    d2.wait()
    _zero_ref(acc_v, _NPAD)

    bufs = ((qr0_v, kr0_v, sem0), (qr1_v, kr1_v, sem1))

    def _off(c):
        return jnp.minimum(loff + c * _CH, _SPAN - _CH)

    def _gather(c, p):
        qr, kr, sem = bufs[p]
        o = _off(c)
        pltpu.async_copy(q_hbm.at[sidx_v.at[pl.ds(o, _CH)]], qr, sem)
        pltpu.async_copy(k_hbm.at[didx_v.at[pl.ds(o, _CH)]], kr, sem)

    _gather(0, 0)
    lane = jnp.arange(_L, dtype=jnp.int32)

    def pair_body(gi, _):
        for p in range(2):
            c = gi * 2 + p
            qr, kr, sem = bufs[p]

            @pl.when(c + 1 < _LOOP_CH)
            def _():
                _gather(c + 1, 1 - p)

            o = _off(c)
            pltpu.make_async_copy(
                q_hbm.at[sidx_v.at[pl.ds(o, _CH)]], qr, sem).wait()
            pltpu.make_async_copy(
                k_hbm.at[didx_v.at[pl.ds(o, _CH)]], kr, sem).wait()

            in_range = c < cnt
            smask = jnp.full((_L,), in_range)
            lax.fori_loop(0, _GROUPS,
                          _rowwise_groups(qr, kr, sidx_v, expall_v,
                                          acc_v, smask, lane, c, o), 0)
        return 0

    lax.fori_loop(0, _LOOP_CH // 2, pair_body, 0)

    pltpu.sync_copy(acc_v, shared_sp.at[sid])
    plsc.subcore_barrier()

    nbase = sid * _NODES_PER_TILE
    pltpu.sync_copy(shared_sp.at[:, pl.ds(nbase, _NODES_PER_TILE)], red_v)

    def add_body(j, _):
        sl = pl.ds(j * _L, _L)
        s = red_v[0, sl]
        for r in range(1, _NS):
            s = s + red_v[r, sl]
        tot_v[sl] = s
        return 0

    lax.fori_loop(0, _NODES_PER_TILE // _L, add_body, 0)

    @pl.when(cid == 0)
    def _():
        pltpu.sync_copy(tot_v, p0_hbm.at[pl.ds(nbase, _NODES_PER_TILE)])

    @pl.when(cid == 1)
    def _():
        pltpu.sync_copy(tot_v, p1_hbm.at[pl.ds(nbase, _NODES_PER_TILE)])

    plsc.subcore_barrier()

    @pl.when(sid == 0)
    def _():
        flag_v[pl.ds(0, _L)] = jnp.ones((_L,), jnp.float32)
        pltpu.sync_copy(flag_v, flag_hbm.at[cid])

    def poll_cond(s):
        return s < 15.5

    def poll_body(s):
        pltpu.sync_copy(flag_hbm.at[1 - cid], flag_v)
        return jnp.sum(flag_v[pl.ds(0, _L)])

    lax.while_loop(poll_cond, poll_body, jnp.float32(0.0))

    da = pltpu.async_copy(p0_hbm, acc_v, sem0)
    db = pltpu.async_copy(p1_hbm, sum2_v, sem1)
    da.wait()
    db.wait()

    def comb_body(j, _):
        for u in range(4):
            sl = pl.ds((j * 4 + u) * _L, _L)
            acc_v[sl] = acc_v[sl] + sum2_v[sl]
        return 0

    lax.fori_loop(0, _NPAD // (4 * _L), comb_body, 0)

    def div_body(gg, _):
        for u in range(2):
            g = gg * 2 + u
            sl = pl.ds(g * _L, _L)
            srcv = sidx_v[pl.ds(loff + g * _L, _L)]
            sv = plsc.load_gather(acc_v, [srcv])
            sum2_v[sl] = expall_v[sl] / sv
        return 0

    lax.fori_loop(0, cnt * _GROUPS // 2, div_body, 0)

    pltpu.sync_copy(sum2_v.at[pl.ds(0, main_words)],
                    out_hbm.at[pl.ds(e0, main_words)])

    @pl.when(cnt == _BASE_CNT + 1)
    def _():
        pltpu.sync_copy(sum2_v.at[pl.ds(main_words, _CH)],
                        out_hbm.at[pl.ds(e0 + main_words, _CH)])


def _rowwise_groups(qr, kr, sidx_v, expall_v, acc_v, smask, lane, c, o):
    def group_body(g, carry):
        dots = jnp.zeros((_L,), jnp.float32)
        for e in range(_L):
            prod = jnp.zeros((_L,), jnp.float32)
            row = g * _L + e
            for j in range(_FQK // (2 * _L)):
                sl = pl.ds(j * 2 * _L, 2 * _L)
                pp = qr[row, sl] * kr[row, sl]
                pa, pb = plsc.unpack(
                    pp, format=plsc.PackFormat.INTERLEAVED,
                    preferred_element_type=jnp.float32)
                prod = prod + pa + pb
            dots = jnp.where(lane == e, jnp.sum(prod), dots)
        ev = jnp.exp(dots)
        expall_v[pl.ds(c * _CH + g * _L, _L)] = ev
        srcv = sidx_v[pl.ds(o + g * _L, _L)]
        plsc.addupdate_scatter(acc_v, [srcv], ev, mask=smask)
        return carry

    return group_body


def _edge_kernel(q, k, ei):
    mesh = plsc.VectorSubcoreMesh(core_axis_name="c", subcore_axis_name="s")
    kfn = pl.kernel(
        _edge_body,
        out_type=(
            jax.ShapeDtypeStruct((_E,), jnp.float32),
            jax.ShapeDtypeStruct((_NPAD,), jnp.float32),
            jax.ShapeDtypeStruct((_NPAD,), jnp.float32),
            jax.ShapeDtypeStruct((_NC, _L), jnp.float32),
        ),
        mesh=mesh,
        compiler_params=_SC_PARAMS,
        scratch_types=(
            pltpu.VMEM((_SPAN,), jnp.int32),
            pltpu.VMEM((_SPAN,), jnp.int32),
            pltpu.VMEM((_CH, _FQK), jnp.bfloat16),
            pltpu.VMEM((_CH, _FQK), jnp.bfloat16),
            pltpu.VMEM((_CH, _FQK), jnp.bfloat16),
            pltpu.VMEM((_CH, _FQK), jnp.bfloat16),
            pltpu.VMEM((_SPAN,), jnp.float32),
            pltpu.VMEM((_NPAD,), jnp.float32),
            pltpu.VMEM((_NPAD,), jnp.float32),
            pltpu.VMEM((_NS, _NODES_PER_TILE), jnp.float32),
            pltpu.VMEM((_NODES_PER_TILE,), jnp.float32),
            pltpu.VMEM((_L,), jnp.float32),
            pltpu.VMEM_SHARED((_NS, _NPAD), jnp.float32),
            pltpu.SemaphoreType.DMA,
            pltpu.SemaphoreType.DMA,
        ),
    )
    out, _, _, _ = kfn(q, k, ei)
    return out


def kernel(x, batch, ei, W):
    del batch
    q, k = _project(x, W)
    return _edge_kernel(q, k, ei)


# speedup vs baseline: 1.4062x; 1.0230x over previous
import jax
import jax.numpy as jnp
from jax import lax
from jax.experimental import pallas as pl
from jax.experimental.pallas import tpu as pltpu
from jax.experimental.pallas import tpu_sc as plsc

_FIN = 128
_FQK = 64
_N = 10000
_E = 320000
_NPAD = 10240
_NC, _NS, _L = 2, 16, 16
_NW = _NC * _NS
_CH = 128
_NCHUNK = _E // _CH
_BASE_CNT = _NCHUNK // _NW
_EXTRA = _NCHUNK - _BASE_CNT * _NW
_LOOP_CH = 80
_SPAN = _LOOP_CH * _CH

_NODES_PER_TILE = _NPAD // _NS
_GROUPS = _CH // _L


def _proj_body(x_ref, w_ref, q_ref, k_ref):
    qk = jnp.dot(x_ref[...], w_ref[...], preferred_element_type=jnp.float32)
    scale = float(_FQK) ** (-0.5)
    q_ref[...] = (qk[:, :_FQK] * scale).astype(jnp.bfloat16)
    k_ref[...] = qk[:, _FQK:].astype(jnp.bfloat16)


def _project(x, W):
    return pl.pallas_call(
        _proj_body,
        out_shape=(
            jax.ShapeDtypeStruct((_N, _FQK), jnp.bfloat16),
            jax.ShapeDtypeStruct((_N, _FQK), jnp.bfloat16),
        ),
    )(x, W)


def _worker_span(wid):
    base = wid * _BASE_CNT + jnp.minimum(wid, _EXTRA)
    cnt = _BASE_CNT + jnp.where(wid < _EXTRA, 1, 0)
    return base, cnt


def _zero_ref(ref, nwords):
    zeros = jnp.zeros((_L,), jnp.float32)

    def body(i, _):
        for u in range(4):
            ref[pl.ds((i * 4 + u) * _L, _L)] = zeros
        return 0

    lax.fori_loop(0, nwords // (4 * _L), body, 0)


_SC_PARAMS = pltpu.CompilerParams(
    needs_layout_passes=False, use_tc_tiling_on_sc=False)


def _edge_body(q_hbm, k_hbm, ei_hbm,
               out_hbm, p0_hbm, p1_hbm, flag_hbm,
               sidx_v, didx_v, qr0_v, kr0_v, qr1_v, kr1_v,
               expall_v, acc_v, sum2_v, red_v, tot_v, flag_v,
               shared_sp, sem0, sem1):
    cid = lax.axis_index("c")
    sid = lax.axis_index("s")
    wid = sid * _NC + cid
    base, cnt = _worker_span(wid)
    e0 = base * _CH
    main_words = _BASE_CNT * _CH

    @pl.when(sid == 0)
    def _():
        flag_v[pl.ds(0, _L)] = jnp.zeros((_L,), jnp.float32)
        pltpu.sync_copy(flag_v, flag_hbm.at[cid])

    e0c = jnp.minimum(e0, _E - _SPAN)
    loff = e0 - e0c
    d1 = pltpu.async_copy(ei_hbm.at[0, pl.ds(e0c, _SPAN)], sidx_v, sem0)
    d2 = pltpu.async_copy(ei_hbm.at[1, pl.ds(e0c, _SPAN)], didx_v, sem0)
    d1.wait()
    d2.wait()
    _zero_ref(acc_v, _NPAD)

    bufs = ((qr0_v, kr0_v, sem0), (qr1_v, kr1_v, sem1))

    def _off(c):
        return jnp.minimum(loff + c * _CH, _SPAN - _CH)

    def _gather(c, p):
        qr, kr, sem = bufs[p]
        o = _off(c)
        pltpu.async_copy(q_hbm.at[sidx_v.at[pl.ds(o, _CH)]], qr, sem)
        pltpu.async_copy(k_hbm.at[didx_v.at[pl.ds(o, _CH)]], kr, sem)

    _gather(0, 0)
    lane = jnp.arange(_L, dtype=jnp.int32)

    def pair_body(gi, _):
        for p in range(2):
            c = gi * 2 + p
            qr, kr, sem = bufs[p]

            @pl.when(c + 1 < _LOOP_CH)
            def _():
                _gather(c + 1, 1 - p)

            o = _off(c)
            pltpu.make_async_copy(
                q_hbm.at[sidx_v.at[pl.ds(o, _CH)]], qr, sem).wait()
            pltpu.make_async_copy(
                k_hbm.at[didx_v.at[pl.ds(o, _CH)]], kr, sem).wait()

            in_range = c < cnt
            smask = jnp.full((_L,), in_range)
            lax.fori_loop(0, _GROUPS,
                          _rowwise_groups(qr, kr, sidx_v, expall_v,
                                          acc_v, smask, lane, c, o), 0)
        return 0

    lax.fori_loop(0, _LOOP_CH // 2, pair_body, 0)

    pltpu.sync_copy(acc_v, shared_sp.at[sid])
    plsc.subcore_barrier()

    nbase = sid * _NODES_PER_TILE
    pltpu.sync_copy(shared_sp.at[:, pl.ds(nbase, _NODES_PER_TILE)], red_v)

    def add_body(j, _):
        sl = pl.ds(j * _L, _L)
        s = red_v[0, sl]
        for r in range(1, _NS):
            s = s + red_v[r, sl]
        tot_v[sl] = s
        return 0

    lax.fori_loop(0, _NODES_PER_TILE // _L, add_body, 0)

    @pl.when(cid == 0)
    def _():
        pltpu.sync_copy(tot_v, p0_hbm.at[pl.ds(nbase, _NODES_PER_TILE)])

    @pl.when(cid == 1)
    def _():
        pltpu.sync_copy(tot_v, p1_hbm.at[pl.ds(nbase, _NODES_PER_TILE)])

    plsc.subcore_barrier()

    @pl.when(sid == 0)
    def _():
        flag_v[pl.ds(0, _L)] = jnp.ones((_L,), jnp.float32)
        pltpu.sync_copy(flag_v, flag_hbm.at[cid])

    def poll_cond(s):
        return s < 15.5

    def poll_body(s):
        pltpu.sync_copy(flag_hbm.at[1 - cid], flag_v)
        return jnp.sum(flag_v[pl.ds(0, _L)])

    lax.while_loop(poll_cond, poll_body, jnp.float32(0.0))

    da = pltpu.async_copy(p0_hbm, acc_v, sem0)
    db = pltpu.async_copy(p1_hbm, sum2_v, sem1)
    da.wait()
    db.wait()

    one = jnp.ones((_L,), jnp.float32)

    def comb_body(j, _):
        for u in range(4):
            sl = pl.ds((j * 4 + u) * _L, _L)
            acc_v[sl] = one / (acc_v[sl] + sum2_v[sl])
        return 0

    lax.fori_loop(0, _NPAD // (4 * _L), comb_body, 0)

    def div_body(gg, _):
        for u in range(2):
            g = gg * 2 + u
            sl = pl.ds(g * _L, _L)
            srcv = sidx_v[pl.ds(loff + g * _L, _L)]
            sv = plsc.load_gather(acc_v, [srcv])
            sum2_v[sl] = expall_v[sl] * sv
        return 0

    lax.fori_loop(0, cnt * _GROUPS // 2, div_body, 0)

    pltpu.sync_copy(sum2_v.at[pl.ds(0, main_words)],
                    out_hbm.at[pl.ds(e0, main_words)])

    @pl.when(cnt == _BASE_CNT + 1)
    def _():
        pltpu.sync_copy(sum2_v.at[pl.ds(main_words, _CH)],
                        out_hbm.at[pl.ds(e0 + main_words, _CH)])


def _rowwise_groups(qr, kr, sidx_v, expall_v, acc_v, smask, lane, c, o):
    def group_body(g, carry):
        dots = jnp.zeros((_L,), jnp.float32)
        for e in range(_L):
            prod = jnp.zeros((_L,), jnp.float32)
            row = g * _L + e
            for j in range(_FQK // (2 * _L)):
                sl = pl.ds(j * 2 * _L, 2 * _L)
                pp = qr[row, sl] * kr[row, sl]
                pa, pb = plsc.unpack(
                    pp, format=plsc.PackFormat.INTERLEAVED,
                    preferred_element_type=jnp.float32)
                prod = prod + pa + pb
            dots = jnp.where(lane == e, jnp.sum(prod), dots)
        ev = jnp.exp(dots)
        expall_v[pl.ds(c * _CH + g * _L, _L)] = ev
        srcv = sidx_v[pl.ds(o + g * _L, _L)]
        plsc.addupdate_scatter(acc_v, [srcv], ev, mask=smask)
        return carry

    return group_body


def _edge_kernel(q, k, ei):
    mesh = plsc.VectorSubcoreMesh(core_axis_name="c", subcore_axis_name="s")
    kfn = pl.kernel(
        _edge_body,
        out_type=(
            jax.ShapeDtypeStruct((_E,), jnp.float32),
            jax.ShapeDtypeStruct((_NPAD,), jnp.float32),
            jax.ShapeDtypeStruct((_NPAD,), jnp.float32),
            jax.ShapeDtypeStruct((_NC, _L), jnp.float32),
        ),
        mesh=mesh,
        compiler_params=_SC_PARAMS,
        scratch_types=(
            pltpu.VMEM((_SPAN,), jnp.int32),
            pltpu.VMEM((_SPAN,), jnp.int32),
            pltpu.VMEM((_CH, _FQK), jnp.bfloat16),
            pltpu.VMEM((_CH, _FQK), jnp.bfloat16),
            pltpu.VMEM((_CH, _FQK), jnp.bfloat16),
            pltpu.VMEM((_CH, _FQK), jnp.bfloat16),
            pltpu.VMEM((_SPAN,), jnp.float32),
            pltpu.VMEM((_NPAD,), jnp.float32),
            pltpu.VMEM((_NPAD,), jnp.float32),
            pltpu.VMEM((_NS, _NODES_PER_TILE), jnp.float32),
            pltpu.VMEM((_NODES_PER_TILE,), jnp.float32),
            pltpu.VMEM((_L,), jnp.float32),
            pltpu.VMEM_SHARED((_NS, _NPAD), jnp.float32),
            pltpu.SemaphoreType.DMA,
            pltpu.SemaphoreType.DMA,
        ),
    )
    out, _, _, _ = kfn(q, k, ei)
    return out


def kernel(x, batch, ei, W):
    del batch
    q, k = _project(x, W)
    return _edge_kernel(q, k, ei)
